# baseline (speedup 1.0000x reference)
import jax
import jax.numpy as jnp
from jax.experimental import pallas as pl
from jax.experimental.pallas import tpu as pltpu

B = 64
P = 576
D = 384
BB = 8


def _add_body(pos_ref, table_ref, patches_ref, out_ref, emb_ref):
    @pl.when(pl.program_id(0) == 0)
    def _():
        pos = pos_ref[...]
        onehot = (pos[:, None] == jax.lax.broadcasted_iota(jnp.int32, (P, P), 1))
        emb_ref[...] = jnp.dot(onehot.astype(jnp.float32), table_ref[...],
                               preferred_element_type=jnp.float32)

    out_ref[...] = patches_ref[...] + emb_ref[...][None]


_tc_kernel = pl.pallas_call(
    _add_body,
    grid=(B // BB,),
    in_specs=[
        pl.BlockSpec((P,), lambda i: (0,)),
        pl.BlockSpec((P, D), lambda i: (0, 0)),
        pl.BlockSpec((BB, P, D), lambda i: (i, 0, 0)),
    ],
    out_specs=pl.BlockSpec((BB, P, D), lambda i: (i, 0, 0)),
    out_shape=jax.ShapeDtypeStruct((B, P, D), jnp.float32),
    scratch_shapes=[pltpu.VMEM((P, D), jnp.float32)],
)


def kernel(encoded_patches, pos_embedding_table, positions):
    return _tc_kernel(positions, pos_embedding_table, encoded_patches)


# speedup vs baseline: 1.0721x; 1.0721x over previous
import jax
import jax.numpy as jnp
from jax.experimental import pallas as pl
from jax.experimental.pallas import tpu as pltpu

B = 64
P = 576
D = 384
BB = 16


def _add_body(pos_ref, table_ref, patches_ref, out_ref, emb_ref):
    @pl.when(pl.program_id(0) == 0)
    def _():
        pos = pos_ref[...]
        onehot = (pos[:, None] == jax.lax.broadcasted_iota(jnp.int32, (P, P), 1))
        emb_ref[...] = jnp.dot(onehot.astype(jnp.float32), table_ref[...],
                               preferred_element_type=jnp.float32)

    out_ref[...] = patches_ref[...] + emb_ref[...][None]


_tc_kernel = pl.pallas_call(
    _add_body,
    grid=(B // BB,),
    in_specs=[
        pl.BlockSpec((P,), lambda i: (0,)),
        pl.BlockSpec((P, D), lambda i: (0, 0)),
        pl.BlockSpec((BB, P, D), lambda i: (i, 0, 0)),
    ],
    out_specs=pl.BlockSpec((BB, P, D), lambda i: (i, 0, 0)),
    out_shape=jax.ShapeDtypeStruct((B, P, D), jnp.float32),
    scratch_shapes=[pltpu.VMEM((P, D), jnp.float32)],
)


def kernel(encoded_patches, pos_embedding_table, positions):
    return _tc_kernel(positions, pos_embedding_table, encoded_patches)
